# baseline (device time: 21161 ns/iter reference)
import jax
import jax.numpy as jnp
from jax import lax
from jax.experimental import pallas as pl
from jax.experimental.pallas import tpu as pltpu

N_DEV = 8
BLOCK_M = 1024


def kernel(x):
    m_per, n = x.shape
    nblocks = m_per // BLOCK_M

    def body(x_ref, o_ref, pbuf, rbuf, ssems, rsems):
        i = pl.program_id(0)
        me = lax.axis_index("i")

        xb = x_ref[...]
        m = jnp.max(xb, axis=0, keepdims=True)
        rows = lax.broadcasted_iota(jnp.int32, xb.shape, 0)
        li = jnp.min(
            jnp.where(xb == m, rows, BLOCK_M), axis=0, keepdims=True
        )
        gidx = li.astype(jnp.float32) + (
            me.astype(jnp.float32) * float(m_per)
            + i.astype(jnp.float32) * float(BLOCK_M)
        )

        @pl.when(i == 0)
        def _():
            pbuf[0:1, :] = m
            pbuf[1:2, :] = gidx

        @pl.when(i > 0)
        def _():
            take = m > pbuf[0:1, :]
            pbuf[0:1, :] = jnp.where(take, m, pbuf[0:1, :])
            pbuf[1:2, :] = jnp.where(take, gidx, pbuf[1:2, :])

        @pl.when(i == nblocks - 1)
        def _():
            barrier = pltpu.get_barrier_semaphore()
            for k in range(1, N_DEV):
                pl.semaphore_signal(
                    barrier, inc=1, device_id=((me + k) % N_DEV,),
                    device_id_type=pl.DeviceIdType.MESH,
                )
            pl.semaphore_wait(barrier, N_DEV - 1)

            rdmas = []
            for k in range(1, N_DEV):
                rdma = pltpu.make_async_remote_copy(
                    src_ref=pbuf,
                    dst_ref=rbuf.at[k - 1],
                    send_sem=ssems.at[k - 1],
                    recv_sem=rsems.at[k - 1],
                    device_id=((me + k) % N_DEV,),
                    device_id_type=pl.DeviceIdType.MESH,
                )
                rdma.start()
                rdmas.append(rdma)
            for rdma in rdmas:
                rdma.wait()

            acc_v = pbuf[0:1, :]
            acc_i = pbuf[1:2, :]
            for k in range(1, N_DEV):
                v = rbuf[k - 1, 0:1, :]
                ix = rbuf[k - 1, 1:2, :]
                take = (v > acc_v) | ((v == acc_v) & (ix < acc_i))
                acc_v = jnp.where(take, v, acc_v)
                acc_i = jnp.where(take, ix, acc_i)
            o_ref[0:1, :] = acc_v
            o_ref[1:2, :] = acc_i

    return pl.pallas_call(
        body,
        grid=(nblocks,),
        in_specs=[pl.BlockSpec((BLOCK_M, n), lambda i: (i, 0))],
        out_specs=pl.BlockSpec((2, n), lambda i: (0, 0)),
        out_shape=jax.ShapeDtypeStruct((2, n), jnp.float32),
        scratch_shapes=[
            pltpu.VMEM((2, n), jnp.float32),
            pltpu.VMEM((N_DEV - 1, 2, n), jnp.float32),
            pltpu.SemaphoreType.DMA((N_DEV - 1,)),
            pltpu.SemaphoreType.DMA((N_DEV - 1,)),
        ],
        compiler_params=pltpu.CompilerParams(collective_id=0),
    )(x)


# device time: 12251 ns/iter; 1.7273x vs baseline; 1.7273x over previous
import os

import jax
import jax.numpy as jnp
from jax import lax
from jax.experimental import pallas as pl
from jax.experimental.pallas import tpu as pltpu

N_DEV = 8
BLOCK_M = 1024
_VARIANT = os.environ.get("KVARIANT", "eqmin")


def kernel(x):
    m_per, n = x.shape
    nblocks = m_per // BLOCK_M

    def body(x_ref, o_ref, pbuf, rbuf, ssems, rsems):
        i = pl.program_id(0)
        me = lax.axis_index("i")

        xb = x_ref[...]
        m = jnp.max(xb, axis=0, keepdims=True)
        if _VARIANT == "maxonly":
            li = jnp.zeros((1, xb.shape[1]), jnp.int32)
        elif _VARIANT == "argmax":
            li = jnp.argmax(xb, axis=0).reshape(1, -1).astype(jnp.int32)
        else:
            rows = lax.broadcasted_iota(jnp.int32, xb.shape, 0)
            li = jnp.min(
                jnp.where(xb == m, rows, BLOCK_M), axis=0, keepdims=True
            )
        gidx = li.astype(jnp.float32) + (
            me.astype(jnp.float32) * float(m_per)
            + i.astype(jnp.float32) * float(BLOCK_M)
        )

        @pl.when(i == 0)
        def _():
            pbuf[0:1, :] = m
            pbuf[1:2, :] = gidx

        @pl.when(i > 0)
        def _():
            take = m > pbuf[0:1, :]
            pbuf[0:1, :] = jnp.where(take, m, pbuf[0:1, :])
            pbuf[1:2, :] = jnp.where(take, gidx, pbuf[1:2, :])

        if os.environ.get("LOCAL_ONLY") == "1":
            @pl.when(i == nblocks - 1)
            def _():
                o_ref[...] = pbuf[...]
            return

        @pl.when(i == nblocks - 1)
        def _():
            barrier = pltpu.get_barrier_semaphore()
            for k in range(1, N_DEV):
                pl.semaphore_signal(
                    barrier, inc=1, device_id=((me + k) % N_DEV,),
                    device_id_type=pl.DeviceIdType.MESH,
                )
            pl.semaphore_wait(barrier, N_DEV - 1)

            rdmas = []
            for k in range(1, N_DEV):
                rdma = pltpu.make_async_remote_copy(
                    src_ref=pbuf,
                    dst_ref=rbuf.at[k - 1],
                    send_sem=ssems.at[k - 1],
                    recv_sem=rsems.at[k - 1],
                    device_id=((me + k) % N_DEV,),
                    device_id_type=pl.DeviceIdType.MESH,
                )
                rdma.start()
                rdmas.append(rdma)
            for rdma in rdmas:
                rdma.wait()

            acc_v = pbuf[0:1, :]
            acc_i = pbuf[1:2, :]
            for k in range(1, N_DEV):
                v = rbuf[k - 1, 0:1, :]
                ix = rbuf[k - 1, 1:2, :]
                take = (v > acc_v) | ((v == acc_v) & (ix < acc_i))
                acc_v = jnp.where(take, v, acc_v)
                acc_i = jnp.where(take, ix, acc_i)
            o_ref[0:1, :] = acc_v
            o_ref[1:2, :] = acc_i

    return pl.pallas_call(
        body,
        grid=(nblocks,),
        in_specs=[pl.BlockSpec((BLOCK_M, n), lambda i: (i, 0))],
        out_specs=pl.BlockSpec((2, n), lambda i: (0, 0)),
        out_shape=jax.ShapeDtypeStruct((2, n), jnp.float32),
        scratch_shapes=[
            pltpu.VMEM((2, n), jnp.float32),
            pltpu.VMEM((N_DEV - 1, 2, n), jnp.float32),
            pltpu.SemaphoreType.DMA((N_DEV - 1,)),
            pltpu.SemaphoreType.DMA((N_DEV - 1,)),
        ],
        compiler_params=(
            None
            if os.environ.get("LOCAL_ONLY") == "1"
            else pltpu.CompilerParams(collective_id=0)
        ),
    )(x)
